# baseline (speedup 1.0000x reference)
import jax
import jax.numpy as jnp
from jax import lax
from jax.experimental import pallas as pl
from jax.experimental.pallas import tpu as pltpu

N_ROWS = 8192
DIM = 256
S = 8
SUB = N_ROWS // S

_IMAX = 2147483647


def _body(x_ref, w_hbm, o_ref, buf, *sems):
    copies = [
        pltpu.make_async_copy(
            w_hbm.at[pl.ds(i * SUB, SUB), :],
            buf.at[pl.ds(i * SUB, SUB), :],
            sems[i],
        )
        for i in range(S)
    ]
    for cp in copies:
        cp.start()
    for cp in copies:
        cp.wait()

    xv = x_ref[...].reshape(1, 1, DIM)
    w3 = buf[pl.ds(0, 512), :].reshape(4, 128, DIM)
    d = jnp.sum((xv - w3) ** 2, axis=2)
    gmin = jnp.min(d)
    gidx = (
        lax.broadcasted_iota(jnp.int32, d.shape, 0) * 128
        + lax.broadcasted_iota(jnp.int32, d.shape, 1)
    )
    o_ref[0] = jnp.min(jnp.where(d == gmin, gidx, jnp.int32(_IMAX)))


@jax.jit
def kernel(x, weights):
    out = pl.pallas_call(
        _body,
        in_specs=[
            pl.BlockSpec(memory_space=pltpu.VMEM),
            pl.BlockSpec(memory_space=pltpu.MemorySpace.HBM),
        ],
        out_specs=pl.BlockSpec(memory_space=pltpu.SMEM),
        out_shape=jax.ShapeDtypeStruct((1,), jnp.int32),
        scratch_shapes=(
            [pltpu.VMEM((N_ROWS, DIM), jnp.float32)]
            + [pltpu.SemaphoreType.DMA for _ in range(S)]
        ),
    )(x.reshape(1, DIM), weights)
    return out[0]


# speedup vs baseline: 1.0725x; 1.0725x over previous
import functools

import jax
import jax.numpy as jnp
from jax import lax
from jax.experimental import pallas as pl
from jax.experimental.pallas import tpu as pltpu

N_ROWS = 8192
DIM = 256
BLK = 4096
GRID = N_ROWS // BLK
MB = BLK // 128

_IMAX = 2147483647


def _body(x_ref, w_ref, o_ref, dacc):
    i = pl.program_id(0)
    xv = x_ref[...].reshape(1, 1, DIM)
    w3 = w_ref[...].reshape(MB, 128, DIM)
    dacc[pl.ds(i * MB, MB), :] = jnp.sum((xv - w3) ** 2, axis=2)

    @pl.when(i == GRID - 1)
    def _():
        dist = dacc[...]
        gmin = jnp.min(dist)
        gidx = (
            lax.broadcasted_iota(jnp.int32, dist.shape, 0) * 128
            + lax.broadcasted_iota(jnp.int32, dist.shape, 1)
        )
        o_ref[0] = jnp.min(jnp.where(dist == gmin, gidx, jnp.int32(_IMAX)))


@jax.jit
def kernel(x, weights):
    out = pl.pallas_call(
        _body,
        grid=(GRID,),
        in_specs=[
            pl.BlockSpec((1, DIM), lambda i: (0, 0)),
            pl.BlockSpec((BLK, DIM), lambda i: (i, 0)),
        ],
        out_specs=pl.BlockSpec(memory_space=pltpu.SMEM),
        out_shape=jax.ShapeDtypeStruct((1,), jnp.int32),
        scratch_shapes=[
            pltpu.VMEM((N_ROWS // 128, 128), jnp.float32),
        ],
    )(x.reshape(1, DIM), weights)
    return out[0]
